# Initial kernel scaffold; baseline (speedup 1.0000x reference)
import jax
import jax.numpy as jnp
from jax.experimental import pallas as pl


def kernel(x1, x2, weight, smooth_scales):
    raise NotImplementedError("write your pallas kernel here")


# speedup vs baseline: 1.2647x; 1.2647x over previous
import jax
import jax.numpy as jnp
from jax.experimental import pallas as pl
from jax.experimental.pallas import tpu as pltpu

_EPS = 1e-6
_BLOCK_ROWS = 256


def _fused_body(x1_ref, x2_ref, w_ref, ss_ref,
                x_ref, y_ref, q1_ref, s1_ref, q3_ref, s3_ref):
    x = x1_ref[...] + x2_ref[...]
    x_ref[...] = x
    inv = jax.lax.rsqrt(jnp.mean(x * x, axis=-1, keepdims=True) + _EPS)
    y = x * inv * w_ref[...]
    y_ref[...] = y

    s = y * ss_ref[...]
    a1 = jnp.max(jnp.abs(s), axis=-1, keepdims=True) / 127.0
    q1_ref[...] = jnp.round(s / a1).astype(jnp.int8)
    s1_ref[...] = a1

    a3 = jnp.max(jnp.abs(y), axis=-1, keepdims=True) / 127.0
    q3_ref[...] = jnp.round(y / a3).astype(jnp.int8)
    s3_ref[...] = a3


def kernel(x1, x2, weight, smooth_scales):
    b, ssz, h = x1.shape
    rows = b * ssz
    x1f = x1.reshape(rows, h)
    x2f = x2.reshape(rows, h)
    w2 = weight.reshape(1, h)
    sm2 = smooth_scales.reshape(1, h)

    br = _BLOCK_ROWS
    grid = (rows // br,)
    row_spec = pl.BlockSpec((br, h), lambda i: (i, 0))
    vec_spec = pl.BlockSpec((1, h), lambda i: (0, 0))
    scale_spec = pl.BlockSpec((br, 1), lambda i: (i, 0))

    xout, y, q1, s1, q3, s3 = pl.pallas_call(
        _fused_body,
        grid=grid,
        in_specs=[row_spec, row_spec, vec_spec, vec_spec],
        out_specs=[row_spec, row_spec, row_spec, scale_spec,
                   row_spec, scale_spec],
        out_shape=[
            jax.ShapeDtypeStruct((rows, h), jnp.float32),
            jax.ShapeDtypeStruct((rows, h), jnp.float32),
            jax.ShapeDtypeStruct((rows, h), jnp.int8),
            jax.ShapeDtypeStruct((rows, 1), jnp.float32),
            jax.ShapeDtypeStruct((rows, h), jnp.int8),
            jax.ShapeDtypeStruct((rows, 1), jnp.float32),
        ],
        compiler_params=pltpu.CompilerParams(
            dimension_semantics=("parallel",),
            vmem_limit_bytes=56 * 1024 * 1024,
        ),
        name="fused_add_rmsnorm_dualquant",
    )(x1f, x2f, w2, sm2)

    x3d = xout.reshape(b, ssz, h)
    return (q1.reshape(b, ssz, h), x3d, s1.reshape(b, ssz), y, x3d,
            q1, s1.reshape(rows), x3d, q3, s3)


# speedup vs baseline: 1.2980x; 1.0263x over previous
import jax
import jax.numpy as jnp
from jax.experimental import pallas as pl
from jax.experimental.pallas import tpu as pltpu

_EPS = 1e-6
_BLOCK_ROWS = 256


def _fused_body(x1_ref, x2_ref, w_ref, ss_ref,
                x_ref, y_ref, q1_ref, s1_ref, q3_ref, s3_ref):
    x = x1_ref[...] + x2_ref[...]
    x_ref[...] = x
    inv = jax.lax.rsqrt(jnp.mean(x * x, axis=-1, keepdims=True) + _EPS)
    y = x * inv * w_ref[...]
    y_ref[...] = y

    s = y * ss_ref[...]
    a1 = jnp.max(jnp.abs(s), axis=-1, keepdims=True) / 127.0
    q1_ref[...] = jnp.round(s / a1).astype(jnp.int8)
    s1_ref[...] = a1.reshape(1, 1, -1)

    a3 = jnp.max(jnp.abs(y), axis=-1, keepdims=True) / 127.0
    q3_ref[...] = jnp.round(y / a3).astype(jnp.int8)
    s3_ref[...] = a3.reshape(1, 1, -1)


def kernel(x1, x2, weight, smooth_scales):
    b, ssz, h = x1.shape
    rows = b * ssz
    x1f = x1.reshape(rows, h)
    x2f = x2.reshape(rows, h)
    w2 = weight.reshape(1, h)
    sm2 = smooth_scales.reshape(1, h)

    br = _BLOCK_ROWS
    grid = (rows // br,)
    row_spec = pl.BlockSpec((br, h), lambda i: (i, 0))
    vec_spec = pl.BlockSpec((1, h), lambda i: (0, 0))
    scale_spec = pl.BlockSpec((1, 1, br), lambda i: (i, 0, 0))

    xout, y, q1, s1, q3, s3 = pl.pallas_call(
        _fused_body,
        grid=grid,
        in_specs=[row_spec, row_spec, vec_spec, vec_spec],
        out_specs=[row_spec, row_spec, row_spec, scale_spec,
                   row_spec, scale_spec],
        out_shape=[
            jax.ShapeDtypeStruct((rows, h), jnp.float32),
            jax.ShapeDtypeStruct((rows, h), jnp.float32),
            jax.ShapeDtypeStruct((rows, h), jnp.int8),
            jax.ShapeDtypeStruct((rows // br, 1, br), jnp.float32),
            jax.ShapeDtypeStruct((rows, h), jnp.int8),
            jax.ShapeDtypeStruct((rows // br, 1, br), jnp.float32),
        ],
        compiler_params=pltpu.CompilerParams(
            dimension_semantics=("parallel",),
            vmem_limit_bytes=56 * 1024 * 1024,
        ),
        name="fused_add_rmsnorm_dualquant",
    )(x1f, x2f, w2, sm2)

    x3d = xout.reshape(b, ssz, h)
    s1f = s1.reshape(rows)
    return (q1.reshape(b, ssz, h), x3d, s1f.reshape(b, ssz), y, x3d,
            q1, s1f, x3d, q3, s3.reshape(rows, 1))


# speedup vs baseline: 1.3157x; 1.0137x over previous
import jax
import jax.numpy as jnp
from jax.experimental import pallas as pl
from jax.experimental.pallas import tpu as pltpu

_EPS = 1e-6
_BLOCK_ROWS = 256


def _fused_body(x1_ref, x2_ref, w_ref, ss_ref,
                q1_3d_ref, x_3d_ref, s1_2d_ref, y_ref, q1_ref, s1_ref,
                q3_ref, s3_ref):
    i = pl.program_id(0)
    x = x1_ref[...] + x2_ref[...]
    x_3d_ref[...] = x[None]
    inv = jax.lax.rsqrt(jnp.mean(x * x, axis=-1, keepdims=True) + _EPS)
    y = x * inv * w_ref[...]
    y_ref[...] = y

    br = x.shape[0]
    s = y * ss_ref[...]
    a1 = jnp.max(jnp.abs(s), axis=-1, keepdims=True) / 127.0
    q1 = jnp.round(s / a1).astype(jnp.int8)
    q1_3d_ref[...] = q1[None]
    q1_ref[...] = q1
    a1_lane = a1.reshape(-1)
    blocks_per_batch = s1_2d_ref.shape[1] // br
    s1_2d_ref[i // blocks_per_batch, pl.ds((i % blocks_per_batch) * br, br)] = a1_lane
    s1_ref[pl.ds(i * br, br)] = a1_lane

    a3 = jnp.max(jnp.abs(y), axis=-1, keepdims=True) / 127.0
    q3_ref[...] = jnp.round(y / a3).astype(jnp.int8)
    s3_ref[pl.ds(i * br, br), :] = a3


def kernel(x1, x2, weight, smooth_scales):
    b, ssz, h = x1.shape
    rows = b * ssz
    x1f = x1.reshape(rows, h)
    x2f = x2.reshape(rows, h)
    w2 = weight.reshape(1, h)
    sm2 = smooth_scales.reshape(1, h)

    br = _BLOCK_ROWS
    grid = (rows // br,)
    bpb = ssz // br
    row_spec = pl.BlockSpec((br, h), lambda i: (i, 0))
    vec_spec = pl.BlockSpec((1, h), lambda i: (0, 0))
    row3d_spec = pl.BlockSpec((1, br, h), lambda i: (i // bpb, i % bpb, 0))

    q1_3d, x_3d, s1_2d, y, q1, s1, q3, s3 = pl.pallas_call(
        _fused_body,
        grid=grid,
        in_specs=[row_spec, row_spec, vec_spec, vec_spec],
        out_specs=[
            row3d_spec,
            row3d_spec,
            pl.BlockSpec((b, ssz), lambda i: (0, 0)),
            row_spec,
            row_spec,
            pl.BlockSpec((rows,), lambda i: (0,)),
            row_spec,
            pl.BlockSpec((rows, 1), lambda i: (0, 0)),
        ],
        out_shape=[
            jax.ShapeDtypeStruct((b, ssz, h), jnp.int8),
            jax.ShapeDtypeStruct((b, ssz, h), jnp.float32),
            jax.ShapeDtypeStruct((b, ssz), jnp.float32),
            jax.ShapeDtypeStruct((rows, h), jnp.float32),
            jax.ShapeDtypeStruct((rows, h), jnp.int8),
            jax.ShapeDtypeStruct((rows,), jnp.float32),
            jax.ShapeDtypeStruct((rows, h), jnp.int8),
            jax.ShapeDtypeStruct((rows, 1), jnp.float32),
        ],
        compiler_params=pltpu.CompilerParams(
            dimension_semantics=("arbitrary",),
            vmem_limit_bytes=56 * 1024 * 1024,
        ),
        name="fused_add_rmsnorm_dualquant",
    )(x1f, x2f, w2, sm2)

    return (q1_3d, x_3d, s1_2d, y, x_3d, q1, s1, x_3d, q3, s3)


# speedup vs baseline: 1.7630x; 1.3399x over previous
import jax
import jax.numpy as jnp
from jax.experimental import pallas as pl
from jax.experimental.pallas import tpu as pltpu

_EPS = 1e-6
_BLOCK_ROWS = 256


def _fused_body(x1_ref, x2_ref, w_ref, ss_ref,
                q1_3d_ref, xa_ref, xb_ref, xc_ref, s1_2d_ref, y_ref,
                q1_ref, s1_ref, q3_ref, s3_ref):
    i = pl.program_id(0)
    x = x1_ref[...] + x2_ref[...]
    xa_ref[...] = x[None]
    xb_ref[...] = x[None]
    xc_ref[...] = x[None]
    inv = jax.lax.rsqrt(jnp.mean(x * x, axis=-1, keepdims=True) + _EPS)
    y = x * inv * w_ref[...]
    y_ref[...] = y

    br = x.shape[0]
    s = y * ss_ref[...]
    a1 = jnp.max(jnp.abs(s), axis=-1, keepdims=True) / 127.0
    q1 = jnp.round(s / a1).astype(jnp.int8)
    q1_3d_ref[...] = q1[None]
    q1_ref[...] = q1
    a1_lane = a1.reshape(-1)
    blocks_per_batch = s1_2d_ref.shape[1] // br
    s1_2d_ref[i // blocks_per_batch, pl.ds((i % blocks_per_batch) * br, br)] = a1_lane
    s1_ref[pl.ds(i * br, br)] = a1_lane

    a3 = jnp.max(jnp.abs(y), axis=-1, keepdims=True) / 127.0
    q3_ref[...] = jnp.round(y / a3).astype(jnp.int8)
    s3_ref[...] = a3.reshape(1, 1, -1)


def kernel(x1, x2, weight, smooth_scales):
    b, ssz, h = x1.shape
    rows = b * ssz
    x1f = x1.reshape(rows, h)
    x2f = x2.reshape(rows, h)
    w2 = weight.reshape(1, h)
    sm2 = smooth_scales.reshape(1, h)

    br = _BLOCK_ROWS
    grid = (rows // br,)
    bpb = ssz // br
    row_spec = pl.BlockSpec((br, h), lambda i: (i, 0))
    vec_spec = pl.BlockSpec((1, h), lambda i: (0, 0))
    row3d_spec = pl.BlockSpec((1, br, h), lambda i: (i // bpb, i % bpb, 0))

    q1_3d, xa, xb, xc, s1_2d, y, q1, s1, q3, s3 = pl.pallas_call(
        _fused_body,
        grid=grid,
        in_specs=[row_spec, row_spec, vec_spec, vec_spec],
        out_specs=[
            row3d_spec,
            row3d_spec,
            row3d_spec,
            row3d_spec,
            pl.BlockSpec((b, ssz), lambda i: (0, 0)),
            row_spec,
            row_spec,
            pl.BlockSpec((rows,), lambda i: (0,)),
            row_spec,
            pl.BlockSpec((1, 1, br), lambda i: (i, 0, 0)),
        ],
        out_shape=[
            jax.ShapeDtypeStruct((b, ssz, h), jnp.int8),
            jax.ShapeDtypeStruct((b, ssz, h), jnp.float32),
            jax.ShapeDtypeStruct((b, ssz, h), jnp.float32),
            jax.ShapeDtypeStruct((b, ssz, h), jnp.float32),
            jax.ShapeDtypeStruct((b, ssz), jnp.float32),
            jax.ShapeDtypeStruct((rows, h), jnp.float32),
            jax.ShapeDtypeStruct((rows, h), jnp.int8),
            jax.ShapeDtypeStruct((rows,), jnp.float32),
            jax.ShapeDtypeStruct((rows, h), jnp.int8),
            jax.ShapeDtypeStruct((rows // br, 1, br), jnp.float32),
        ],
        compiler_params=pltpu.CompilerParams(
            dimension_semantics=("arbitrary",),
            vmem_limit_bytes=60 * 1024 * 1024,
        ),
        name="fused_add_rmsnorm_dualquant",
    )(x1f, x2f, w2, sm2)

    return (q1_3d, xa, s1_2d, y, xb, q1, s1, xc, q3, s3.reshape(rows, 1))
